# Initial kernel scaffold; baseline (speedup 1.0000x reference)
import jax
import jax.numpy as jnp
from jax.experimental import pallas as pl


def kernel(x, edge_index, edge_attr, batch, Wf, bf, Ws, bs, W1, b1, W2, b2):
    raise NotImplementedError("write your pallas kernel here")


# speedup vs baseline: 2.7997x; 2.7997x over previous
import functools

import jax
import jax.numpy as jnp
from jax import lax
from jax.experimental import pallas as pl
from jax.experimental.pallas import tpu as pltpu
from jax.experimental.pallas import tpu_sc as plsc

N = 10000
E = 320000
D = 128
DE = 16
H = 128
G = 64

_info = plsc.get_sparse_core_info()
NC = _info.num_cores
NS = _info.num_subcores
NW = NC * NS
EPW = E // NW
CH = 80
NCHUNK = EPW // CH
ZR = 40
WTILES = 10
RPS = N // WTILES

_mesh = plsc.VectorSubcoreMesh(core_axis_name="c", subcore_axis_name="s")


def _tables_body(x_ref, wd_ref, ws_ref, b_ref, td_ref, ts_ref):
    xb = x_ref[...]
    td_ref[...] = (
        jnp.dot(xb, wd_ref[...], preferred_element_type=jnp.float32) + b_ref[...]
    )
    ts_ref[...] = jnp.dot(xb, ws_ref[...], preferred_element_type=jnp.float32)


def _tables(x, wd, wsr, bcat):
    bm = 1000
    return pl.pallas_call(
        _tables_body,
        grid=(N // bm,),
        in_specs=[
            pl.BlockSpec((bm, D), lambda i: (i, 0)),
            pl.BlockSpec((D, 2 * D), lambda i: (0, 0)),
            pl.BlockSpec((D, 2 * D), lambda i: (0, 0)),
            pl.BlockSpec((1, 2 * D), lambda i: (0, 0)),
        ],
        out_specs=[
            pl.BlockSpec((bm, 2 * D), lambda i: (i, 0)),
            pl.BlockSpec((bm, 2 * D), lambda i: (i, 0)),
        ],
        out_shape=[
            jax.ShapeDtypeStruct((N, 2 * D), jnp.float32),
            jax.ShapeDtypeStruct((N, 2 * D), jnp.float32),
        ],
    )(x, wd, wsr, bcat)


@functools.partial(
    pl.kernel,
    out_type=jax.ShapeDtypeStruct((E, 2 * D), jnp.float32),
    mesh=_mesh,
    scratch_types=[
        pltpu.VMEM((NCHUNK, CH), jnp.int32),
        pltpu.VMEM((NCHUNK, CH), jnp.int32),
        pltpu.VMEM((CH, 2 * D), jnp.float32),
        pltpu.VMEM((CH, 2 * D), jnp.float32),
        pltpu.SemaphoreType.DMA,
        pltpu.SemaphoreType.DMA,
    ],
)
def _gather_k(td_hbm, ts_hbm, dst_hbm, src_hbm, g_hbm, idxd, idxs, bufd, bufs,
              semd, sems):
    wid = lax.axis_index("s") * NC + lax.axis_index("c")
    pltpu.sync_copy(dst_hbm.at[wid], idxd)
    pltpu.sync_copy(src_hbm.at[wid], idxs)

    def chunk(c, _):
        cpd = pltpu.make_async_copy(td_hbm.at[idxd.at[c]], bufd, semd)
        cps = pltpu.make_async_copy(ts_hbm.at[idxs.at[c]], bufs, sems)
        cpd.start()
        cps.start()
        cpd.wait()
        cps.wait()

        def add_row(r, _):
            for j in range(16):
                s = pl.ds(j * 16, 16)
                bufd[r, s] = bufd[r, s] + bufs[r, s]
            return 0

        lax.fori_loop(0, CH, add_row, 0, unroll=2)
        pltpu.sync_copy(bufd, g_hbm.at[pl.ds(wid * EPW + c * CH, CH)])
        return 0

    lax.fori_loop(0, NCHUNK, chunk, 0)


def _msg_body(g_ref, ea_ref, we_ref, msg_ref):
    zz = g_ref[...] + jnp.dot(
        ea_ref[...], we_ref[...], preferred_element_type=jnp.float32
    )
    msg_ref[...] = jax.nn.sigmoid(zz[:, :D]) * jax.nn.softplus(zz[:, D:])


def _edge_msg(g, ea, we):
    bm = 2000
    return pl.pallas_call(
        _msg_body,
        grid=(E // bm,),
        in_specs=[
            pl.BlockSpec((bm, 2 * D), lambda i: (i, 0)),
            pl.BlockSpec((bm, DE), lambda i: (i, 0)),
            pl.BlockSpec((DE, 2 * D), lambda i: (0, 0)),
        ],
        out_specs=pl.BlockSpec((bm, D), lambda i: (i, 0)),
        out_shape=jax.ShapeDtypeStruct((E, D), jnp.float32),
    )(g, ea, we)


@functools.partial(
    pl.kernel,
    out_type=jax.ShapeDtypeStruct((NC, N, D), jnp.float32),
    mesh=_mesh,
    scratch_types=[
        pltpu.VMEM((NCHUNK, CH), jnp.int32),
        pltpu.VMEM((CH, D), jnp.float32),
        pltpu.VMEM((ZR, D), jnp.float32),
        pltpu.VMEM_SHARED((N, D), jnp.float32),
        pltpu.SemaphoreType.DMA,
    ],
)
def _scatter_k(msg_hbm, dst_hbm, aggp_hbm, idxd, mbuf, zbuf, aggsh, sem):
    cid = lax.axis_index("c")
    sid = lax.axis_index("s")
    wid = sid * NC + cid
    pltpu.sync_copy(dst_hbm.at[wid], idxd)

    def zrow(i, _):
        for j in range(D // 16):
            zbuf[i, pl.ds(j * 16, 16)] = jnp.zeros((16,), jnp.float32)
        return 0

    lax.fori_loop(0, ZR, zrow, 0)

    @pl.when(sid < WTILES)
    def _init():
        for t in range(RPS // ZR):
            pltpu.sync_copy(zbuf, aggsh.at[pl.ds(sid * RPS + t * ZR, ZR)])

    plsc.subcore_barrier()

    def chunk(c, _):
        pltpu.sync_copy(msg_hbm.at[pl.ds(wid * EPW + c * CH, CH)], mbuf)
        pltpu.sync_copy(mbuf, aggsh.at[idxd.at[c]], add=True)
        return 0

    lax.fori_loop(0, NCHUNK, chunk, 0)
    plsc.subcore_barrier()

    @pl.when(sid < WTILES)
    def _writeout():
        pltpu.sync_copy(
            aggsh.at[pl.ds(sid * RPS, RPS)],
            aggp_hbm.at[cid, pl.ds(sid * RPS, RPS)],
        )


def _final_body(x_ref, ap_ref, b_ref, w1_ref, b1_ref, w2_ref, b2_ref, out_ref):
    h = jax.nn.sigmoid(x_ref[...] + ap_ref[0] + ap_ref[1])
    h = jax.nn.sigmoid(
        jnp.dot(h, w1_ref[...], preferred_element_type=jnp.float32) + b1_ref[...]
    )
    oh = (
        b_ref[...] == lax.broadcasted_iota(jnp.int32, (N, G), 1)
    ).astype(jnp.float32)
    pooled = lax.dot_general(
        oh, h, (((0,), (0,)), ((), ())), preferred_element_type=jnp.float32
    )
    out = jnp.dot(pooled, w2_ref[...], preferred_element_type=jnp.float32)
    out_ref[...] = jnp.maximum(out + b2_ref[...], 0.0)


def _final(x, aggp, batch2d, w1, b1, w2, b2):
    return pl.pallas_call(
        _final_body,
        out_shape=jax.ShapeDtypeStruct((G, 1), jnp.float32),
    )(x, aggp, batch2d, w1, b1, w2, b2)


def kernel(x, edge_index, edge_attr, batch, Wf, bf, Ws, bs, W1, b1, W2, b2):
    src = edge_index[0]
    dst = edge_index[1]
    wd = jnp.concatenate([Wf[:D], Ws[:D]], axis=1)
    wsr = jnp.concatenate([Wf[D : 2 * D], Ws[D : 2 * D]], axis=1)
    we = jnp.concatenate([Wf[2 * D :], Ws[2 * D :]], axis=1)
    bcat = jnp.concatenate([bf, bs]).reshape(1, 2 * D)
    td, ts = _tables(x, wd, wsr, bcat)
    dst3 = dst.reshape(NW, NCHUNK, CH)
    src3 = src.reshape(NW, NCHUNK, CH)
    g = _gather_k(td, ts, dst3, src3)
    msg = _edge_msg(g, edge_attr, we)
    aggp = _scatter_k(msg, dst3)
    return _final(
        x,
        aggp,
        batch.reshape(N, 1),
        W1,
        b1.reshape(1, H),
        W2,
        b2.reshape(1, 1),
    )
